# Initial kernel scaffold; baseline (speedup 1.0000x reference)
import jax
import jax.numpy as jnp
from jax.experimental import pallas as pl


def kernel(node_feat, edge_index, edge_feat, We, be, Wpre, bpre, Wpost, bpost, Wlin, blin, gamma, beta):
    raise NotImplementedError("write your pallas kernel here")


# speedup vs baseline: 2.6651x; 2.6651x over previous
import functools

import jax
import jax.numpy as jnp
from jax import lax
from jax.experimental import pallas as pl
from jax.experimental.pallas import tpu as pltpu
from jax.experimental.pallas import tpu_sc as plsc

N = 10000
E = 320000
D = 128
DE = 4
L = 4
DELTA = 2.5

HR = 80
NSUB = 128
SPW = NSUB // 32
NP = NSUB * HR
CH = 48
EB = 512
EPAD = 512
EP = E + EPAD
NB = 1000
NV = D // 16

_F32 = jnp.float32
_HI = jax.lax.Precision.HIGHEST


def _pre_body(x_ref, wi_ref, wj_ref, a_ref, b_ref):
    x = x_ref[...]
    a_ref[...] = jnp.dot(x, wi_ref[...], preferred_element_type=_F32,
                         precision=_HI)
    b_ref[...] = jnp.dot(x, wj_ref[...], preferred_element_type=_F32,
                         precision=_HI)


def _pre(x, wi, wj):
    return pl.pallas_call(
        _pre_body,
        grid=(N // NB,),
        in_specs=[
            pl.BlockSpec((NB, D), lambda i: (i, 0)),
            pl.BlockSpec((D, D), lambda i: (0, 0)),
            pl.BlockSpec((D, D), lambda i: (0, 0)),
        ],
        out_specs=[
            pl.BlockSpec((NB, D), lambda i: (i, 0)),
            pl.BlockSpec((NB, D), lambda i: (i, 0)),
        ],
        out_shape=[jax.ShapeDtypeStruct((N, D), _F32)] * 2,
    )(x, wi, wj)


def _edgec_body(ef_ref, we_ref, wpe_ref, be_ref, bpre_ref, c_ref):
    wpe = wpe_ref[...]
    w4 = jnp.dot(we_ref[...], wpe, preferred_element_type=_F32, precision=_HI)
    bias = (jnp.dot(be_ref[...], wpe, preferred_element_type=_F32,
                    precision=_HI) + bpre_ref[...])
    c_ref[...] = jnp.dot(ef_ref[...], w4, preferred_element_type=_F32,
                         precision=_HI) + bias


def _edgec(ef_p, we_l, wpe_l, be_l, bpre_l):
    return pl.pallas_call(
        _edgec_body,
        grid=(EP // EB,),
        in_specs=[
            pl.BlockSpec((EB, DE), lambda i: (i, 0)),
            pl.BlockSpec((DE, D), lambda i: (0, 0)),
            pl.BlockSpec((D, D), lambda i: (0, 0)),
            pl.BlockSpec((1, D), lambda i: (0, 0)),
            pl.BlockSpec((1, D), lambda i: (0, 0)),
        ],
        out_specs=pl.BlockSpec((EB, D), lambda i: (i, 0)),
        out_shape=jax.ShapeDtypeStruct((EP, D), _F32),
    )(ef_p, we_l, wpe_l, be_l, bpre_l)


def _sc_extract(stv, i):
    return stv[pl.ds(i, 16)][0]


def _agg_body(a_hbm, b_hbm, c_hbm, dst_hbm, src_hbm, st_hbm,
              s_hbm, q_hbm, mx_hbm, mn_hbm, dg_hbm,
              ta, tsum, tsq, tmx, tmn, tdg, dsti, srci, bbuf, cbuf, stv,
              semb):
    wid = lax.axis_index("s") * 2 + lax.axis_index("c")
    pltpu.sync_copy(st_hbm, stv)
    zero16 = jnp.zeros((16,), _F32)
    ninf16 = jnp.full((16,), -3.0e38, _F32)
    pinf16 = jnp.full((16,), 3.0e38, _F32)
    ones16 = jnp.full((16,), 1.0, _F32)

    def sub_body(uu, _unused):
        i = wid * SPW + uu
        lo_n = i * HR
        lo_e = _sc_extract(stv, i)
        hi_e = _sc_extract(stv, i + 1)

        pltpu.sync_copy(a_hbm.at[pl.ds(lo_n, HR)], ta)

        def zr(r, _2):
            for j in range(NV):
                sl = pl.ds(j * 16, 16)
                tsum[r, sl] = zero16
                tsq[r, sl] = zero16
                tmx[r, sl] = ninf16
                tmn[r, sl] = pinf16
            tdg[r, pl.ds(0, 16)] = zero16
            return 0

        lax.fori_loop(0, HR, zr, 0)

        e0a = (lo_e // 8) * 8
        nch = (hi_e - e0a + CH - 1) // CH

        def chunk(ci, _2):
            e0 = e0a + ci * CH
            pltpu.sync_copy(dst_hbm.at[pl.ds(e0, CH)], dsti.at[pl.ds(0, CH)])
            pltpu.sync_copy(src_hbm.at[pl.ds(e0, CH)], srci)
            cpb = pltpu.async_copy(b_hbm.at[srci], bbuf, semb)
            pltpu.sync_copy(c_hbm.at[pl.ds(e0, CH)], cbuf)
            cpb.wait()
            klo = jnp.maximum(lo_e - e0, 0)
            khi = jnp.minimum(hi_e - e0, CH)

            def edge(k, _3):
                d = dsti[pl.ds(k, 16)][0]
                r = d - lo_n
                for j in range(NV):
                    sl = pl.ds(j * 16, 16)
                    t = bbuf[k, sl] + cbuf[k, sl]
                    tsum[r, sl] += t
                    tsq[r, sl] += t * t
                    tmx[r, sl] = jnp.maximum(tmx[r, sl], t)
                    tmn[r, sl] = jnp.minimum(tmn[r, sl], t)
                tdg[r, pl.ds(0, 16)] += ones16
                return 0

            lax.fori_loop(klo, khi, edge, 0)
            return 0

        lax.fori_loop(0, nch, chunk, 0)

        def fix(r, _2):
            cnt = tdg[r, pl.ds(0, 16)][0]
            has = cnt > 0.0
            for j in range(NV):
                sl = pl.ds(j * 16, 16)
                av = ta[r, sl]
                s_t = tsum[r, sl]
                tsum[r, sl] = s_t + cnt * av
                tsq[r, sl] = tsq[r, sl] + av * (2.0 * s_t + cnt * av)
                tmx[r, sl] = jnp.where(has, tmx[r, sl] + av, 0.0)
                tmn[r, sl] = jnp.where(has, tmn[r, sl] + av, 0.0)
            return 0

        lax.fori_loop(0, HR, fix, 0)

        pltpu.sync_copy(tsum, s_hbm.at[pl.ds(lo_n, HR)])
        pltpu.sync_copy(tsq, q_hbm.at[pl.ds(lo_n, HR)])
        pltpu.sync_copy(tmx, mx_hbm.at[pl.ds(lo_n, HR)])
        pltpu.sync_copy(tmn, mn_hbm.at[pl.ds(lo_n, HR)])
        pltpu.sync_copy(tdg, dg_hbm.at[pl.ds(lo_n, HR)])
        return 0

    lax.fori_loop(0, SPW, sub_body, 0)


def _agg(a_pad, b_pad, c, dst_p, src_p, starts_p):
    mesh = plsc.VectorSubcoreMesh(core_axis_name="c", subcore_axis_name="s")
    fn = pl.kernel(
        _agg_body,
        mesh=mesh,
        out_type=[jax.ShapeDtypeStruct((NP, D), _F32)] * 4
        + [jax.ShapeDtypeStruct((NP, 16), _F32)],
        scratch_types=[
            pltpu.VMEM((HR, D), _F32),
            pltpu.VMEM((HR, D), _F32),
            pltpu.VMEM((HR, D), _F32),
            pltpu.VMEM((HR, D), _F32),
            pltpu.VMEM((HR, D), _F32),
            pltpu.VMEM((HR, 16), _F32),
            pltpu.VMEM((CH + 16,), jnp.int32),
            pltpu.VMEM((CH,), jnp.int32),
            pltpu.VMEM((CH, D), _F32),
            pltpu.VMEM((CH, D), _F32),
            pltpu.VMEM((152,), jnp.int32),
            pltpu.SemaphoreType.DMA,
        ],
    )
    return fn(a_pad, b_pad, c, dst_p, src_p, starts_p)


CHP = 400
EPW = E // 32


def _perm_body(pk_hbm, pos_hbm, out_hbm, pkbuf, pbuf, sem0):
    wid = lax.axis_index("s") * 2 + lax.axis_index("c")
    base = wid * EPW
    for ci in range(EPW // CHP):
        e0 = base + ci * CHP
        pltpu.sync_copy(pk_hbm.at[pl.ds(e0, CHP)], pkbuf)
        pltpu.sync_copy(pos_hbm.at[pl.ds(e0, CHP)], pbuf)
        pltpu.async_copy(pkbuf, out_hbm.at[pbuf], sem0).wait()


def _perm(packed, pos):
    mesh = plsc.VectorSubcoreMesh(core_axis_name="c", subcore_axis_name="s")
    fn = pl.kernel(
        _perm_body,
        mesh=mesh,
        out_type=jax.ShapeDtypeStruct((E, D), _F32),
        scratch_types=[
            pltpu.VMEM((CHP, D), _F32),
            pltpu.VMEM((CHP,), jnp.int32),
            pltpu.SemaphoreType.DMA,
        ],
    )
    return fn(packed, pos)

def _post_body(x_ref, s_ref, q_ref, mx_ref, mn_ref, dg_ref,
               wx_ref, w1_ref, w2_ref, w3_ref, bpost_ref, wlin_ref, blin_ref,
               h_ref, st_ref):
    deg = dg_ref[...][:, 0:1]
    degc = jnp.maximum(deg, 1.0)
    s = s_ref[...]
    mean = s / degc
    var = q_ref[...] / degc - mean * mean
    std = jnp.sqrt(jnp.maximum(var, 0.0) + 1e-5)
    agg = jnp.concatenate([mean, mx_ref[...], mn_ref[...], std, var, s],
                          axis=1)
    logd = jnp.log(degc + 1.0)
    amp = logd * (1.0 / DELTA)
    att = DELTA / logd
    p1 = jnp.dot(agg, w1_ref[...], preferred_element_type=_F32)
    p2 = jnp.dot(agg, w2_ref[...], preferred_element_type=_F32)
    p3 = jnp.dot(agg, w3_ref[...], preferred_element_type=_F32)
    out = (jnp.dot(x_ref[...], wx_ref[...], preferred_element_type=_F32)
           + p1 + amp * p2 + att * p3 + bpost_ref[...])
    out = jnp.dot(out, wlin_ref[...], preferred_element_type=_F32) \
        + blin_ref[...]
    h_ref[...] = out

    @pl.when(pl.program_id(0) == 0)
    def _():
        st_ref[...] = jnp.zeros_like(st_ref)

    st_ref[0:1, :] += jnp.sum(out, axis=0, keepdims=True)
    st_ref[1:2, :] += jnp.sum(out * out, axis=0, keepdims=True)


def _post(x, s, q, mx, mn, dg, wx, w1, w2, w3, bpost_l, wlin_l, blin_l):
    full = lambda shp: pl.BlockSpec(shp, lambda i: (0, 0))
    return pl.pallas_call(
        _post_body,
        grid=(N // NB,),
        in_specs=[
            pl.BlockSpec((NB, D), lambda i: (i, 0)),
            pl.BlockSpec((NB, D), lambda i: (i, 0)),
            pl.BlockSpec((NB, D), lambda i: (i, 0)),
            pl.BlockSpec((NB, D), lambda i: (i, 0)),
            pl.BlockSpec((NB, D), lambda i: (i, 0)),
            pl.BlockSpec((NB, 16), lambda i: (i, 0)),
            full((D, D)), full((6 * D, D)), full((6 * D, D)),
            full((6 * D, D)), full((1, D)), full((D, D)), full((1, D)),
        ],
        out_specs=[
            pl.BlockSpec((NB, D), lambda i: (i, 0)),
            pl.BlockSpec((8, D), lambda i: (0, 0)),
        ],
        out_shape=[
            jax.ShapeDtypeStruct((N, D), _F32),
            jax.ShapeDtypeStruct((8, D), _F32),
        ],
    )(x, s, q, mx, mn, dg, wx, w1, w2, w3, bpost_l, wlin_l, blin_l)


def _bn_body(h_ref, x_ref, st_ref, gamma_ref, beta_ref, o_ref):
    mu = st_ref[0:1, :] * (1.0 / N)
    var = st_ref[1:2, :] * (1.0 / N) - mu * mu
    inv = lax.rsqrt(var + 1e-5)
    hn = gamma_ref[...] * (h_ref[...] - mu) * inv + beta_ref[...]
    o_ref[...] = jnp.maximum(hn, 0.0) + x_ref[...]


def _bn(h, x, stats, gamma_l, beta_l):
    return pl.pallas_call(
        _bn_body,
        grid=(N // NB,),
        in_specs=[
            pl.BlockSpec((NB, D), lambda i: (i, 0)),
            pl.BlockSpec((NB, D), lambda i: (i, 0)),
            pl.BlockSpec((8, D), lambda i: (0, 0)),
            pl.BlockSpec((1, D), lambda i: (0, 0)),
            pl.BlockSpec((1, D), lambda i: (0, 0)),
        ],
        out_specs=pl.BlockSpec((NB, D), lambda i: (i, 0)),
        out_shape=jax.ShapeDtypeStruct((N, D), _F32),
    )(h, x, stats, gamma_l, beta_l)


def kernel(node_feat, edge_index, edge_feat, We, be, Wpre, bpre, Wpost,
           bpost, Wlin, blin, gamma, beta):
    src = edge_index[0].astype(jnp.int32)
    dst = edge_index[1].astype(jnp.int32)
    bucket = dst // HR
    ohf = (bucket[:, None] == jnp.arange(NSUB, dtype=jnp.int32)[None, :])
    ohf = ohf.astype(_F32)
    within = jnp.cumsum(ohf, axis=0)
    rank = jnp.sum(ohf * within, axis=1) - 1.0
    counts = jnp.sum(ohf, axis=0)
    starts_f = jnp.concatenate(
        [jnp.zeros((1,), _F32), jnp.cumsum(counts)])
    posf = jnp.dot(ohf, starts_f[:NSUB], precision=_HI) + rank
    pos = posf.astype(jnp.int32)
    packed = jnp.concatenate(
        [dst.astype(_F32)[:, None], src.astype(_F32)[:, None],
         edge_feat.astype(_F32),
         jnp.zeros((E, D - 2 - DE), _F32)], axis=1)
    packed_s = _perm(packed, pos)
    dst_s = packed_s[:, 0].astype(jnp.int32)
    src_s = packed_s[:, 1].astype(jnp.int32)
    ef_s = packed_s[:, 2:2 + DE]
    src_p = jnp.concatenate([src_s, jnp.zeros((EPAD,), jnp.int32)])
    dst_p = jnp.concatenate([dst_s, jnp.zeros((EPAD,), jnp.int32)])
    ef_p = jnp.concatenate(
        [ef_s, jnp.zeros((EPAD, DE), _F32)], axis=0)
    starts = starts_f.astype(jnp.int32)
    starts_p = jnp.concatenate([starts, jnp.zeros((23,), jnp.int32)])

    x = node_feat
    zpad = jnp.zeros((NP - N, D), _F32)
    for l in range(L):
        wi = Wpre[l, :D, :]
        wj = Wpre[l, D:2 * D, :]
        wpe = Wpre[l, 2 * D:, :]
        a, b = _pre(x, wi, wj)
        c = _edgec(ef_p, We[l], wpe, be[l].reshape(1, D),
                   bpre[l].reshape(1, D))
        a_pad = jnp.concatenate([a, zpad], axis=0)
        b_pad = jnp.concatenate([b, zpad], axis=0)
        s, q, mx, mn, dg = _agg(a_pad, b_pad, c, dst_p, src_p, starts_p)
        h, stats = _post(x, s[:N], q[:N], mx[:N], mn[:N], dg[:N],
                         Wpost[l, :D, :], Wpost[l, D:7 * D, :],
                         Wpost[l, 7 * D:13 * D, :], Wpost[l, 13 * D:, :],
                         bpost[l].reshape(1, D), Wlin[l],
                         blin[l].reshape(1, D))
        x = _bn(h, x, stats, gamma[l].reshape(1, D), beta[l].reshape(1, D))
    return x
